# Initial kernel scaffold; baseline (speedup 1.0000x reference)
import jax
import jax.numpy as jnp
from jax.experimental import pallas as pl


def kernel(x, w_qkv, b_qkv, w_lepe, b_lepe, w_out, b_out):
    raise NotImplementedError("write your pallas kernel here")


# speedup vs baseline: 3.7277x; 3.7277x over previous
import functools

import jax
import jax.numpy as jnp
from jax.experimental import pallas as pl
from jax.experimental.pallas import tpu as pltpu

DIM = 96
NUM_HEADS = 4
HD = DIM // NUM_HEADS
N_WIN = 28
TOPK = 4
RS = 8
RS2 = RS * RS
NREG = N_WIN * N_WIN
SCALE = DIM ** (-0.5)
H = W = 224
BAND = RS
NBAND = H // BAND


def _qkv_kernel(x_ref, w_ref, b_ref,
                qs_ref, ks_ref, vs_ref, vg_ref, qr_ref, kr_ref):
    xf = x_ref[...].reshape(DIM, BAND * W)
    qkv = jax.lax.dot_general(
        w_ref[...], xf, (((1,), (0,)), ((), ())),
        preferred_element_type=jnp.float32) + b_ref[...]
    q = qkv[:DIM].reshape(DIM, BAND, W)
    k = qkv[DIM:2 * DIM].reshape(DIM, BAND, W)
    v = qkv[2 * DIM:].reshape(DIM, BAND, W)
    vg_ref[...] = v

    def to_seq(t):
        t = t.reshape(NUM_HEADS, HD, RS, N_WIN, RS)
        t = t.transpose(0, 3, 2, 4, 1)
        return t.reshape(NUM_HEADS, N_WIN, RS2, HD)

    qs_ref[...] = to_seq(q)
    ks_ref[...] = to_seq(k)
    vs_ref[...] = to_seq(v)

    def to_pool(t):
        m = t.reshape(DIM, RS, N_WIN, RS).mean(axis=(1, 3))
        return m.transpose(1, 0).reshape(1, N_WIN, DIM)

    qr_ref[...] = to_pool(q)
    kr_ref[...] = to_pool(k)


def _route_kernel(qr_ref, kr_ref, idx_ref):
    a = jax.lax.dot_general(
        qr_ref[...], kr_ref[...], (((1,), (1,)), ((), ())),
        preferred_element_type=jnp.float32)
    iota = jax.lax.broadcasted_iota(jnp.int32, (NREG, NREG), 1)
    neg = jnp.float32(jnp.finfo(jnp.float32).min)
    cols = []
    work = a
    for _ in range(TOPK):
        m = jnp.max(work, axis=1, keepdims=True)
        amax = jnp.min(jnp.where(work == m, iota, NREG), axis=1,
                       keepdims=True)
        cols.append(amax)
        work = jnp.where(iota == amax, neg, work)
    idx_ref[...] = jnp.concatenate(cols, axis=1)


def _attn_kernel(idx_ref, q_ref, k0_ref, k1_ref, k2_ref, k3_ref,
                 v0_ref, v1_ref, v2_ref, v3_ref, o_ref):
    del idx_ref
    q = q_ref[:, 0] * SCALE
    kcat = jnp.concatenate(
        [k0_ref[:, 0], k1_ref[:, 0], k2_ref[:, 0], k3_ref[:, 0]], axis=1)
    vcat = jnp.concatenate(
        [v0_ref[:, 0], v1_ref[:, 0], v2_ref[:, 0], v3_ref[:, 0]], axis=1)
    outs = []
    for m in range(NUM_HEADS):
        attn = jax.lax.dot_general(
            q[m], kcat[m], (((1,), (1,)), ((), ())),
            preferred_element_type=jnp.float32)
        attn = attn - jnp.max(attn, axis=1, keepdims=True)
        e = jnp.exp(attn)
        p = e / jnp.sum(e, axis=1, keepdims=True)
        o = jax.lax.dot_general(
            p, vcat[m], (((1,), (0,)), ((), ())),
            preferred_element_type=jnp.float32)
        outs.append(o[None])
    o_ref[...] = jnp.concatenate(outs, axis=0)[:, None]


def _merge_kernel(a_ref, vp_ref, vc_ref, vn_ref,
                  wl_ref, bl_ref, wo_ref, bo_ref, o_ref):
    i = pl.program_id(0)
    a = a_ref[...]
    a = a.reshape(NUM_HEADS, N_WIN, RS, RS, HD)
    a = a.transpose(0, 4, 2, 1, 3)
    a = a.reshape(DIM, BAND, W)
    top = jnp.where(i == 0, 0.0, vp_ref[:, BAND - 1:BAND, :])
    bot = jnp.where(i == NBAND - 1, 0.0, vn_ref[:, 0:1, :])
    ctx = jnp.concatenate([top, vc_ref[...], bot], axis=1)
    zcol = jnp.zeros((DIM, BAND + 2, 1), jnp.float32)
    ctx = jnp.concatenate([zcol, ctx, zcol], axis=2)
    wl = wl_ref[...]
    acc = jnp.zeros((DIM, BAND, W), jnp.float32)
    for dy in range(3):
        for dx in range(3):
            acc = acc + ctx[:, dy:dy + BAND, dx:dx + W] * wl[:, 3 * dy + dx][:, None, None]
    lepe = acc + bl_ref[...][:, :, None]
    merged = (a + lepe).reshape(DIM, BAND * W)
    out = jax.lax.dot_general(
        wo_ref[...], merged, (((1,), (0,)), ((), ())),
        preferred_element_type=jnp.float32) + bo_ref[...]
    o_ref[...] = out.reshape(DIM, BAND, W)


def kernel(x, w_qkv, b_qkv, w_lepe, b_lepe, w_out, b_out):
    f32 = jnp.float32
    x2 = x.reshape(DIM, H, W)
    b2 = b_qkv.reshape(3 * DIM, 1)

    seq_shape = jax.ShapeDtypeStruct((NUM_HEADS, NREG, RS2, HD), f32)
    seq_spec = pl.BlockSpec((NUM_HEADS, N_WIN, RS2, HD), lambda i: (0, i, 0, 0))
    pool_shape = jax.ShapeDtypeStruct((NBAND, N_WIN, DIM), f32)
    pool_spec = pl.BlockSpec((1, N_WIN, DIM), lambda i: (i, 0, 0))
    grid_shape = jax.ShapeDtypeStruct((DIM, H, W), f32)
    grid_spec_b = pl.BlockSpec((DIM, BAND, W), lambda i: (0, i, 0))
    full = lambda shape: pl.BlockSpec(shape, lambda i: (0,) * len(shape))

    qs, ks, vs, vg, qr3, kr3 = pl.pallas_call(
        _qkv_kernel,
        grid=(NBAND,),
        in_specs=[grid_spec_b, full((3 * DIM, DIM)), full((3 * DIM, 1))],
        out_specs=[seq_spec, seq_spec, seq_spec, grid_spec_b,
                   pool_spec, pool_spec],
        out_shape=[seq_shape, seq_shape, seq_shape, grid_shape,
                   pool_shape, pool_shape],
        compiler_params=pltpu.CompilerParams(
            vmem_limit_bytes=100 * 1024 * 1024),
    )(x2, w_qkv, b2)

    qr = qr3.reshape(NREG, DIM)
    kr = kr3.reshape(NREG, DIM)

    idx = pl.pallas_call(
        _route_kernel,
        in_specs=[pl.BlockSpec((NREG, DIM), lambda: (0, 0))] * 2,
        out_specs=pl.BlockSpec((NREG, TOPK), lambda: (0, 0)),
        out_shape=jax.ShapeDtypeStruct((NREG, TOPK), jnp.int32),
    )(qr, kr)

    reg_block = (NUM_HEADS, 1, RS2, HD)
    q_spec = pl.BlockSpec(reg_block, lambda i, idx_ref: (0, i, 0, 0))

    def routed(t):
        return pl.BlockSpec(
            reg_block, lambda i, idx_ref, t=t: (0, idx_ref[i, t], 0, 0))

    attn_out = pl.pallas_call(
        _attn_kernel,
        grid_spec=pltpu.PrefetchScalarGridSpec(
            num_scalar_prefetch=1,
            grid=(NREG,),
            in_specs=[q_spec,
                      routed(0), routed(1), routed(2), routed(3),
                      routed(0), routed(1), routed(2), routed(3)],
            out_specs=q_spec,
        ),
        out_shape=jax.ShapeDtypeStruct((NUM_HEADS, NREG, RS2, HD), f32),
    )(idx, qs, ks, ks, ks, ks, vs, vs, vs, vs)

    out = pl.pallas_call(
        _merge_kernel,
        grid=(NBAND,),
        in_specs=[seq_spec,
                  pl.BlockSpec((DIM, BAND, W),
                               lambda i: (0, jnp.maximum(i - 1, 0), 0)),
                  grid_spec_b,
                  pl.BlockSpec((DIM, BAND, W),
                               lambda i: (0, jnp.minimum(i + 1, NBAND - 1), 0)),
                  full((DIM, 9)), full((DIM, 1)),
                  full((DIM, DIM)), full((DIM, 1))],
        out_specs=grid_spec_b,
        out_shape=grid_shape,
    )(attn_out, vg, vg, vg,
      w_lepe.reshape(DIM, 9), b_lepe.reshape(DIM, 1),
      w_out, b_out.reshape(DIM, 1))

    return out.reshape(1, DIM, H, W)


# speedup vs baseline: 3.8681x; 1.0377x over previous
import functools

import jax
import jax.numpy as jnp
from jax.experimental import pallas as pl
from jax.experimental.pallas import tpu as pltpu

DIM = 96
NUM_HEADS = 4
HD = DIM // NUM_HEADS
N_WIN = 28
TOPK = 4
RS = 8
RS2 = RS * RS
NREG = N_WIN * N_WIN
SCALE = DIM ** (-0.5)
H = W = 224
BAND = RS
NBAND = H // BAND


def _qkv_kernel(x_ref, w_ref, b_ref,
                qs_ref, ks_ref, vs_ref, vg_ref, qr_ref, kr_ref):
    xf = x_ref[...].reshape(DIM, BAND * W)
    qkv = jax.lax.dot_general(
        w_ref[...], xf, (((1,), (0,)), ((), ())),
        preferred_element_type=jnp.float32) + b_ref[...]
    q = qkv[:DIM].reshape(DIM, BAND, W)
    k = qkv[DIM:2 * DIM].reshape(DIM, BAND, W)
    v = qkv[2 * DIM:].reshape(DIM, BAND, W)
    vg_ref[...] = v

    def to_seq(t):
        t = t.reshape(NUM_HEADS, HD, RS, N_WIN, RS)
        t = t.transpose(0, 3, 2, 4, 1)
        return t.reshape(NUM_HEADS, N_WIN, RS2, HD)

    qs_ref[...] = to_seq(q)
    ks_ref[...] = to_seq(k)
    vs_ref[...] = to_seq(v)

    def to_pool(t):
        m = t.reshape(DIM, RS, N_WIN, RS).mean(axis=(1, 3))
        return m.transpose(1, 0).reshape(1, N_WIN, DIM)

    qr_ref[...] = to_pool(q)
    kr_ref[...] = to_pool(k)


def _route_kernel(qr_ref, kr_ref, idx_ref):
    a = jax.lax.dot_general(
        qr_ref[...], kr_ref[...], (((1,), (1,)), ((), ())),
        preferred_element_type=jnp.float32)
    iota = jax.lax.broadcasted_iota(jnp.int32, (NREG, NREG), 1)
    neg = jnp.float32(jnp.finfo(jnp.float32).min)
    cols = []
    work = a
    for _ in range(TOPK):
        m = jnp.max(work, axis=1, keepdims=True)
        amax = jnp.min(jnp.where(work == m, iota, NREG), axis=1,
                       keepdims=True)
        cols.append(amax)
        work = jnp.where(iota == amax, neg, work)
    idx_ref[...] = jnp.concatenate(cols, axis=1)


QB = 8


def _attn_kernel(idx_ref, q_ref, *refs):
    del idx_ref
    krefs = refs[:TOPK * QB]
    vrefs = refs[TOPK * QB:2 * TOPK * QB]
    o_ref = refs[-1]
    for j in range(QB):
        q = q_ref[:, j] * SCALE
        kcat = jnp.concatenate(
            [krefs[TOPK * j + t][:, 0] for t in range(TOPK)], axis=1)
        vcat = jnp.concatenate(
            [vrefs[TOPK * j + t][:, 0] for t in range(TOPK)], axis=1)
        outs = []
        for m in range(NUM_HEADS):
            attn = jax.lax.dot_general(
                q[m], kcat[m], (((1,), (1,)), ((), ())),
                preferred_element_type=jnp.float32)
            attn = attn - jnp.max(attn, axis=1, keepdims=True)
            e = jnp.exp(attn)
            p = e / jnp.sum(e, axis=1, keepdims=True)
            o = jax.lax.dot_general(
                p, vcat[m], (((1,), (0,)), ((), ())),
                preferred_element_type=jnp.float32)
            outs.append(o[None])
        o_ref[:, j] = jnp.concatenate(outs, axis=0)


def _merge_kernel(a_ref, vp_ref, vc_ref, vn_ref,
                  wl_ref, bl_ref, wo_ref, bo_ref, o_ref):
    i = pl.program_id(0)
    a = a_ref[...]
    a = a.reshape(NUM_HEADS, N_WIN, RS, RS, HD)
    a = a.transpose(0, 4, 2, 1, 3)
    a = a.reshape(DIM, BAND, W)
    top = jnp.where(i == 0, 0.0, vp_ref[:, BAND - 1:BAND, :])
    bot = jnp.where(i == NBAND - 1, 0.0, vn_ref[:, 0:1, :])
    ctx = jnp.concatenate([top, vc_ref[...], bot], axis=1)
    zcol = jnp.zeros((DIM, BAND + 2, 1), jnp.float32)
    ctx = jnp.concatenate([zcol, ctx, zcol], axis=2)
    wl = wl_ref[...]
    acc = jnp.zeros((DIM, BAND, W), jnp.float32)
    for dy in range(3):
        for dx in range(3):
            acc = acc + ctx[:, dy:dy + BAND, dx:dx + W] * wl[:, 3 * dy + dx][:, None, None]
    lepe = acc + bl_ref[...][:, :, None]
    merged = (a + lepe).reshape(DIM, BAND * W)
    out = jax.lax.dot_general(
        wo_ref[...], merged, (((1,), (0,)), ((), ())),
        preferred_element_type=jnp.float32) + bo_ref[...]
    o_ref[...] = out.reshape(DIM, BAND, W)


def kernel(x, w_qkv, b_qkv, w_lepe, b_lepe, w_out, b_out):
    f32 = jnp.float32
    x2 = x.reshape(DIM, H, W)
    b2 = b_qkv.reshape(3 * DIM, 1)

    seq_shape = jax.ShapeDtypeStruct((NUM_HEADS, NREG, RS2, HD), f32)
    seq_spec = pl.BlockSpec((NUM_HEADS, N_WIN, RS2, HD), lambda i: (0, i, 0, 0))
    pool_shape = jax.ShapeDtypeStruct((NBAND, N_WIN, DIM), f32)
    pool_spec = pl.BlockSpec((1, N_WIN, DIM), lambda i: (i, 0, 0))
    grid_shape = jax.ShapeDtypeStruct((DIM, H, W), f32)
    grid_spec_b = pl.BlockSpec((DIM, BAND, W), lambda i: (0, i, 0))
    full = lambda shape: pl.BlockSpec(shape, lambda i: (0,) * len(shape))

    qs, ks, vs, vg, qr3, kr3 = pl.pallas_call(
        _qkv_kernel,
        grid=(NBAND,),
        in_specs=[grid_spec_b, full((3 * DIM, DIM)), full((3 * DIM, 1))],
        out_specs=[seq_spec, seq_spec, seq_spec, grid_spec_b,
                   pool_spec, pool_spec],
        out_shape=[seq_shape, seq_shape, seq_shape, grid_shape,
                   pool_shape, pool_shape],
        compiler_params=pltpu.CompilerParams(
            vmem_limit_bytes=100 * 1024 * 1024),
    )(x2, w_qkv, b2)

    qr = qr3.reshape(NREG, DIM)
    kr = kr3.reshape(NREG, DIM)

    idx = pl.pallas_call(
        _route_kernel,
        in_specs=[pl.BlockSpec((NREG, DIM), lambda: (0, 0))] * 2,
        out_specs=pl.BlockSpec((NREG, TOPK), lambda: (0, 0)),
        out_shape=jax.ShapeDtypeStruct((NREG, TOPK), jnp.int32),
    )(qr, kr)

    reg_block = (NUM_HEADS, 1, RS2, HD)
    q_spec = pl.BlockSpec((NUM_HEADS, QB, RS2, HD),
                          lambda i, idx_ref: (0, i, 0, 0))

    def routed(j, t):
        return pl.BlockSpec(
            reg_block,
            lambda i, idx_ref, j=j, t=t: (0, idx_ref[i * QB + j, t], 0, 0))

    routed_specs = [routed(j, t) for j in range(QB) for t in range(TOPK)]
    attn_out = pl.pallas_call(
        _attn_kernel,
        grid_spec=pltpu.PrefetchScalarGridSpec(
            num_scalar_prefetch=1,
            grid=(NREG // QB,),
            in_specs=[q_spec] + routed_specs + routed_specs,
            out_specs=q_spec,
        ),
        out_shape=jax.ShapeDtypeStruct((NUM_HEADS, NREG, RS2, HD), f32),
    )(idx, qs, *([ks] * (TOPK * QB)), *([vs] * (TOPK * QB)))

    out = pl.pallas_call(
        _merge_kernel,
        grid=(NBAND,),
        in_specs=[seq_spec,
                  pl.BlockSpec((DIM, BAND, W),
                               lambda i: (0, jnp.maximum(i - 1, 0), 0)),
                  grid_spec_b,
                  pl.BlockSpec((DIM, BAND, W),
                               lambda i: (0, jnp.minimum(i + 1, NBAND - 1), 0)),
                  full((DIM, 9)), full((DIM, 1)),
                  full((DIM, DIM)), full((DIM, 1))],
        out_specs=grid_spec_b,
        out_shape=grid_shape,
    )(attn_out, vg, vg, vg,
      w_lepe.reshape(DIM, 9), b_lepe.reshape(DIM, 1),
      w_out, b_out.reshape(DIM, 1))

    return out.reshape(1, DIM, H, W)


# speedup vs baseline: 7.2859x; 1.8836x over previous
import jax
import jax.numpy as jnp
from jax.experimental import pallas as pl
from jax.experimental.pallas import tpu as pltpu

DIM = 96
NUM_HEADS = 4
HD = DIM // NUM_HEADS
N_WIN = 28
TOPK = 4
RS = 8
RS2 = RS * RS
NREG = N_WIN * N_WIN
SCALE = DIM ** (-0.5)
H = W = 224
BAND = RS
NBAND = H // BAND
QB = 4


def _bf16_dot(a, b, dims):
    return jax.lax.dot_general(a, b, dims,
                               preferred_element_type=jnp.float32)


def _qkv_kernel(x_ref, w_ref, b_ref, qt_ref, kt_ref, vt_ref, pool_ref):
    xb = x_ref[...]
    xf = xb.reshape(DIM, BAND * W)
    qkvt = _bf16_dot(
        xf, w_ref[...], (((0,), (1,)), ((), ()))) + b_ref[...]
    qkvt = qkvt.reshape(BAND, N_WIN, RS, 3 * DIM)
    qt_ref[...] = qkvt[None, :, :, :, :DIM]
    kt_ref[...] = qkvt[None, :, :, :, DIM:2 * DIM]
    vt_ref[...] = qkvt[None, :, :, :, 2 * DIM:]
    pool_ref[...] = qkvt[:, :, :, :2 * DIM].mean(axis=(0, 2))[None]


def _route_kernel(pool_ref, idx_ref):
    qr = pool_ref[..., :DIM].reshape(NREG, DIM)
    kr = pool_ref[..., DIM:].reshape(NREG, DIM)
    a = jax.lax.dot_general(
        qr, kr, (((1,), (1,)), ((), ())),
        preferred_element_type=jnp.float32)
    iota = jax.lax.broadcasted_iota(jnp.int32, (NREG, NREG), 1)
    neg = jnp.float32(jnp.finfo(jnp.float32).min)
    cols = []
    work = a
    for _ in range(TOPK):
        m = jnp.max(work, axis=1, keepdims=True)
        amax = jnp.min(jnp.where(work == m, iota, NREG), axis=1,
                       keepdims=True)
        cols.append(amax)
        work = jnp.where(iota == amax, neg, work)
    idx_ref[...] = jnp.concatenate(cols, axis=1)


def _attn_kernel(idx_ref, q_ref, *refs):
    del idx_ref
    krefs = refs[:TOPK * QB]
    vrefs = refs[TOPK * QB:2 * TOPK * QB]
    o_ref = refs[-1]
    for j in range(QB):
        qreg = q_ref[0, :, j].reshape(RS2, DIM) * SCALE
        kcat = jnp.concatenate(
            [krefs[TOPK * j + t][0, :, 0].reshape(RS2, DIM)
             for t in range(TOPK)], axis=0)
        vcat = jnp.concatenate(
            [vrefs[TOPK * j + t][0, :, 0].reshape(RS2, DIM)
             for t in range(TOPK)], axis=0)
        outs = []
        for m in range(NUM_HEADS):
            sl = slice(HD * m, HD * (m + 1))
            attn = _bf16_dot(
                qreg[:, sl], kcat[:, sl], (((1,), (1,)), ((), ())))
            attn = attn - jnp.max(attn, axis=1, keepdims=True)
            e = jnp.exp(attn)
            o = _bf16_dot(e, vcat[:, sl], (((1,), (0,)), ((), ())))
            o = o / jnp.sum(e, axis=1, keepdims=True)
            outs.append(o)
        oreg = jnp.concatenate(outs, axis=1)
        o_ref[0, :, j] = oreg.reshape(RS, RS, DIM)


def _merge_kernel(a_ref, vp_ref, vc_ref, vn_ref,
                  wl_ref, bl_ref, wo_ref, bo_ref, o_ref):
    i = pl.program_id(0)
    a = a_ref[...].reshape(BAND, W, DIM)
    vc = vc_ref[...].reshape(BAND, W, DIM)
    top = jnp.where(i == 0, 0.0,
                    vp_ref[0, BAND - 1:BAND].reshape(1, W, DIM))
    bot = jnp.where(i == NBAND - 1, 0.0,
                    vn_ref[0, 0:1].reshape(1, W, DIM))
    ctx = jnp.concatenate([top, vc, bot], axis=0)
    zcol = jnp.zeros((BAND + 2, 1, DIM), jnp.float32)
    ctx = jnp.concatenate([zcol, ctx, zcol], axis=1)
    wl = wl_ref[...]
    acc = jnp.zeros((BAND, W, DIM), jnp.float32)
    for dy in range(3):
        for dx in range(3):
            acc = acc + ctx[dy:dy + BAND, dx:dx + W] * wl[3 * dy + dx][None, None]
    lepe = acc + bl_ref[...][None]
    merged = (a + lepe).reshape(BAND * W, DIM)
    out = _bf16_dot(
        wo_ref[...], merged, (((1,), (1,)), ((), ()))) + bo_ref[...]
    o_ref[...] = out.reshape(DIM, BAND, W)


def kernel(x, w_qkv, b_qkv, w_lepe, b_lepe, w_out, b_out):
    f32 = jnp.float32
    x2 = x.reshape(DIM, H, W)
    brow = b_qkv.reshape(1, 3 * DIM)

    seq_shape = jax.ShapeDtypeStruct((NBAND, RS, N_WIN, RS, DIM), f32)
    seq_spec = pl.BlockSpec((1, RS, N_WIN, RS, DIM),
                            lambda i: (i, 0, 0, 0, 0))
    pool_shape = jax.ShapeDtypeStruct((NBAND, N_WIN, 2 * DIM), f32)
    pool_spec = pl.BlockSpec((1, N_WIN, 2 * DIM), lambda i: (i, 0, 0))
    grid_spec_b = pl.BlockSpec((DIM, BAND, W), lambda i: (0, i, 0))
    full = lambda shape: pl.BlockSpec(shape, lambda i: (0,) * len(shape))

    qt, kt, vt, pooled = pl.pallas_call(
        _qkv_kernel,
        grid=(NBAND,),
        in_specs=[grid_spec_b, full((3 * DIM, DIM)), full((1, 3 * DIM))],
        out_specs=[seq_spec, seq_spec, seq_spec, pool_spec],
        out_shape=[seq_shape, seq_shape, seq_shape, pool_shape],
        compiler_params=pltpu.CompilerParams(
            vmem_limit_bytes=100 * 1024 * 1024),
    )(x2, w_qkv, brow)

    fullr = lambda shape: pl.BlockSpec(shape, lambda: (0,) * len(shape))
    idx = pl.pallas_call(
        _route_kernel,
        in_specs=[fullr((NBAND, N_WIN, 2 * DIM))],
        out_specs=fullr((NREG, TOPK)),
        out_shape=jax.ShapeDtypeStruct((NREG, TOPK), jnp.int32),
    )(pooled)

    q_spec = pl.BlockSpec((1, RS, QB, RS, DIM),
                          lambda b, g, idx_ref: (b, 0, g, 0, 0))

    def routed(j, t):
        def imap(b, g, idx_ref, j=j, t=t):
            r = idx_ref[b * N_WIN + g * QB + j, t]
            return (r // N_WIN, 0, r % N_WIN, 0, 0)
        return pl.BlockSpec((1, RS, 1, RS, DIM), imap)

    routed_specs = [routed(j, t) for j in range(QB) for t in range(TOPK)]
    attn_out = pl.pallas_call(
        _attn_kernel,
        grid_spec=pltpu.PrefetchScalarGridSpec(
            num_scalar_prefetch=1,
            grid=(NBAND, N_WIN // QB),
            in_specs=[q_spec] + routed_specs + routed_specs,
            out_specs=q_spec,
        ),
        out_shape=jax.ShapeDtypeStruct((NBAND, RS, N_WIN, RS, DIM), f32),
    )(idx, qt, *([kt] * (TOPK * QB)), *([vt] * (TOPK * QB)))

    out = pl.pallas_call(
        _merge_kernel,
        grid=(NBAND,),
        in_specs=[seq_spec,
                  pl.BlockSpec((1, RS, N_WIN, RS, DIM),
                               lambda i: (jnp.maximum(i - 1, 0), 0, 0, 0, 0)),
                  seq_spec,
                  pl.BlockSpec((1, RS, N_WIN, RS, DIM),
                               lambda i: (jnp.minimum(i + 1, NBAND - 1),
                                          0, 0, 0, 0)),
                  full((9, DIM)), full((1, DIM)),
                  full((DIM, DIM)), full((DIM, 1))],
        out_specs=grid_spec_b,
        out_shape=jax.ShapeDtypeStruct((DIM, H, W), f32),
    )(attn_out, vt, vt, vt,
      w_lepe.reshape(DIM, 9).transpose(1, 0), b_lepe.reshape(1, DIM),
      w_out, b_out.reshape(DIM, 1))

    return out.reshape(1, DIM, H, W)
